# baseline (speedup 1.0000x reference)
import jax
import jax.numpy as jnp
from jax import lax
from jax.experimental import pallas as pl
from jax.experimental.pallas import tpu as pltpu

N, D = 16384, 512
CHR = 128
NCHUNK = N // CHR
NBUF = 8


def _sse_stream(pred_hbm, true_hbm, out_ref, pbuf, tbuf, acc_ref, sems):
    def start(k, slot):
        pltpu.make_async_copy(
            pred_hbm.at[pl.ds(k * CHR, CHR), :],
            pbuf.at[slot],
            sems.at[slot, 0],
        ).start()
        pltpu.make_async_copy(
            true_hbm.at[pl.ds(k * CHR, CHR), :],
            tbuf.at[slot],
            sems.at[slot, 1],
        ).start()

    for k in range(NBUF):
        start(k, k)

    acc_ref[...] = jnp.zeros_like(acc_ref)

    def body(k, _):
        slot = lax.rem(k, NBUF)
        pltpu.make_async_copy(
            pred_hbm.at[pl.ds(0, CHR), :], pbuf.at[slot], sems.at[slot, 0]
        ).wait()
        pltpu.make_async_copy(
            true_hbm.at[pl.ds(0, CHR), :], tbuf.at[slot], sems.at[slot, 1]
        ).wait()
        d = pbuf[slot] - tbuf[slot]
        acc_ref[...] += jnp.sum(d * d, axis=0, keepdims=True)

        @pl.when(k + NBUF < NCHUNK)
        def _():
            nk = k + NBUF

            def dyn_start(hbm, buf, s):
                pltpu.make_async_copy(
                    hbm.at[pl.ds(nk * CHR, CHR), :],
                    buf.at[slot],
                    sems.at[slot, s],
                ).start()

            dyn_start(pred_hbm, pbuf, 0)
            dyn_start(true_hbm, tbuf, 1)

        return 0

    lax.fori_loop(0, NCHUNK, body, 0)
    out_ref[...] = jnp.sum(acc_ref[...]).reshape(1, 1)


def kernel(y_pred, y_true):
    total = pl.pallas_call(
        _sse_stream,
        in_specs=[
            pl.BlockSpec(memory_space=pl.ANY),
            pl.BlockSpec(memory_space=pl.ANY),
        ],
        out_specs=pl.BlockSpec(memory_space=pltpu.MemorySpace.VMEM),
        out_shape=jax.ShapeDtypeStruct((1, 1), jnp.float32),
        scratch_shapes=[
            pltpu.VMEM((NBUF, CHR, D), jnp.float32),
            pltpu.VMEM((NBUF, CHR, D), jnp.float32),
            pltpu.VMEM((1, D), jnp.float32),
            pltpu.SemaphoreType.DMA((NBUF, 2)),
        ],
    )(y_pred, y_true)
    return total[0, 0] / N


# speedup vs baseline: 1.0670x; 1.0670x over previous
import jax
import jax.numpy as jnp
from jax import lax
from jax.experimental import pallas as pl
from jax.experimental.pallas import tpu as pltpu

N, D = 16384, 512
CHR = 256
NCHUNK = N // CHR
NBUF = 8


def _sse_stream(pred_hbm, true_hbm, out_ref, pbuf, tbuf, acc_ref, sems):
    def start(k, slot):
        pltpu.make_async_copy(
            pred_hbm.at[pl.ds(k * CHR, CHR), :],
            pbuf.at[slot],
            sems.at[slot, 0],
        ).start()
        pltpu.make_async_copy(
            true_hbm.at[pl.ds(k * CHR, CHR), :],
            tbuf.at[slot],
            sems.at[slot, 1],
        ).start()

    for k in range(NBUF):
        start(k, k)

    acc_ref[...] = jnp.zeros_like(acc_ref)

    def body(k, _):
        slot = lax.rem(k, NBUF)
        pltpu.make_async_copy(
            pred_hbm.at[pl.ds(0, CHR), :], pbuf.at[slot], sems.at[slot, 0]
        ).wait()
        pltpu.make_async_copy(
            true_hbm.at[pl.ds(0, CHR), :], tbuf.at[slot], sems.at[slot, 1]
        ).wait()
        d = pbuf[slot] - tbuf[slot]
        acc_ref[...] += jnp.sum(d * d, axis=0, keepdims=True)

        @pl.when(k + NBUF < NCHUNK)
        def _():
            nk = k + NBUF

            def dyn_start(hbm, buf, s):
                pltpu.make_async_copy(
                    hbm.at[pl.ds(nk * CHR, CHR), :],
                    buf.at[slot],
                    sems.at[slot, s],
                ).start()

            dyn_start(pred_hbm, pbuf, 0)
            dyn_start(true_hbm, tbuf, 1)

        return 0

    lax.fori_loop(0, NCHUNK, body, 0)
    out_ref[...] = jnp.sum(acc_ref[...]).reshape(1, 1)


def kernel(y_pred, y_true):
    total = pl.pallas_call(
        _sse_stream,
        in_specs=[
            pl.BlockSpec(memory_space=pl.ANY),
            pl.BlockSpec(memory_space=pl.ANY),
        ],
        out_specs=pl.BlockSpec(memory_space=pltpu.MemorySpace.VMEM),
        out_shape=jax.ShapeDtypeStruct((1, 1), jnp.float32),
        scratch_shapes=[
            pltpu.VMEM((NBUF, CHR, D), jnp.float32),
            pltpu.VMEM((NBUF, CHR, D), jnp.float32),
            pltpu.VMEM((1, D), jnp.float32),
            pltpu.SemaphoreType.DMA((NBUF, 2)),
        ],
    )(y_pred, y_true)
    return total[0, 0] / N


# speedup vs baseline: 1.1491x; 1.0770x over previous
import jax
import jax.numpy as jnp
from jax import lax
from jax.experimental import pallas as pl
from jax.experimental.pallas import tpu as pltpu

N, D = 16384, 512
CHR = 256
NCHUNK = N // CHR
NBUF = 8


def _sse_stream(pred_hbm, true_hbm, out_ref, pbuf, tbuf, acc_ref, sems):
    def start(k, slot):
        pltpu.make_async_copy(
            pred_hbm.at[pl.ds(k * CHR, CHR), :],
            pbuf.at[slot],
            sems.at[slot, 0],
        ).start()
        pltpu.make_async_copy(
            true_hbm.at[pl.ds(k * CHR, CHR), :],
            tbuf.at[slot],
            sems.at[slot, 1],
        ).start()

    for k in range(NBUF):
        start(k, k)

    acc_ref[...] = jnp.zeros_like(acc_ref)

    def body(k, _):
        slot = lax.rem(k, NBUF)
        pltpu.make_async_copy(
            pred_hbm.at[pl.ds(0, CHR), :], pbuf.at[slot], sems.at[slot, 0]
        ).wait()
        pltpu.make_async_copy(
            true_hbm.at[pl.ds(0, CHR), :], tbuf.at[slot], sems.at[slot, 1]
        ).wait()
        d = pbuf[slot] - tbuf[slot]
        acc_ref[...] += jnp.sum(d * d, axis=0, keepdims=True)

        @pl.when(k + NBUF < NCHUNK)
        def _():
            nk = k + NBUF

            def dyn_start(hbm, buf, s):
                pltpu.make_async_copy(
                    hbm.at[pl.ds(nk * CHR, CHR), :],
                    buf.at[slot],
                    sems.at[slot, s],
                ).start()

            dyn_start(pred_hbm, pbuf, 0)
            dyn_start(true_hbm, tbuf, 1)

        return 0

    lax.fori_loop(0, NCHUNK, body, 0)
    out_ref[...] = (jnp.sum(acc_ref[...]) * (1.0 / N)).reshape(1, 1)


def kernel(y_pred, y_true):
    total = pl.pallas_call(
        _sse_stream,
        in_specs=[
            pl.BlockSpec(memory_space=pl.ANY),
            pl.BlockSpec(memory_space=pl.ANY),
        ],
        out_specs=pl.BlockSpec(memory_space=pltpu.MemorySpace.VMEM),
        out_shape=jax.ShapeDtypeStruct((1, 1), jnp.float32),
        scratch_shapes=[
            pltpu.VMEM((NBUF, CHR, D), jnp.float32),
            pltpu.VMEM((NBUF, CHR, D), jnp.float32),
            pltpu.VMEM((1, D), jnp.float32),
            pltpu.SemaphoreType.DMA((NBUF, 2)),
        ],
    )(y_pred, y_true)
    return total[0, 0]
